# baseline (speedup 1.0000x reference)
import functools

import jax
import jax.numpy as jnp
import numpy as np
from jax import lax
from jax.experimental import pallas as pl
from jax.experimental.pallas import tpu as pltpu
from jax.experimental.pallas import tpu_sc as plsc

_B = 128
_X = 32768
_N = _B * _X
_TOPK = 64 * _B
_MIN32 = np.int32(-2147483648)

_NC = 2
_NS = 16
_NW = _NC * _NS
_L = 16
_PERW = _N // _NW
_CH = 8192
_NCHUNK = _PERW // _CH


def _sortable(s):
    return s ^ ((s >> 31) | _MIN32)


def _mk_sc_hist(nbins, bin_of, match_of, n_param):
    def body(*refs):
        if n_param:
            x_hbm, p_hbm, out_hbm, buf, hist, merged, pv = refs
        else:
            x_hbm, out_hbm, buf, hist, merged = refs
            pv = None
        wid = lax.axis_index("s") * _NC + lax.axis_index("c")
        base = wid * _PERW
        if n_param:
            pltpu.sync_copy(p_hbm, pv)
        zero16 = jnp.zeros((_L,), jnp.int32)
        ones16 = jnp.ones((_L,), jnp.int32)
        lane = lax.broadcasted_iota(jnp.int32, (_L,), 0)
        laneoff = lane * nbins
        trash = lane + np.int32(nbins * _L)

        def zr(i, carry):
            hist[pl.ds(i * 16, 16)] = zero16
            return carry

        lax.fori_loop(0, nbins * _L // 16, zr, 0, unroll=8)

        def chunk(c, carry):
            pltpu.sync_copy(x_hbm.at[pl.ds(base + c * _CH, _CH)], buf)
            cbase = base + c * _CH

            def sl(j, carry2):
                s = buf[pl.ds(j * 16, 16)]
                ku = _sortable(s)
                idx = (cbase + j * 16) + lax.broadcasted_iota(
                    jnp.int32, (_L,), 0)
                b = bin_of(ku, idx)
                if match_of is None:
                    slot = laneoff + b
                else:
                    m = match_of(ku, idx, pv)
                    slot = jnp.where(m, laneoff + b, trash)
                plsc.addupdate_scatter(hist, [slot], ones16)
                return carry2

            lax.fori_loop(0, _CH // 16, sl, 0, unroll=8)
            return carry

        lax.fori_loop(0, _NCHUNK, chunk, 0)

        def mg(i, carry):
            acc = zero16
            for lane in range(_L):
                acc = acc + hist[pl.ds(lane * nbins + i * 16, 16)]
            merged[pl.ds(i * 16, 16)] = acc
            return carry

        lax.fori_loop(0, nbins // 16, mg, 0)
        pltpu.sync_copy(merged, out_hbm.at[wid])

    mesh = plsc.VectorSubcoreMesh(core_axis_name="c", subcore_axis_name="s")
    scratch = [
        pltpu.VMEM((_CH,), jnp.int32),
        pltpu.VMEM((nbins * _L + _L,), jnp.int32),
        pltpu.VMEM((nbins,), jnp.int32),
    ]
    if n_param:
        scratch.append(pltpu.VMEM((n_param * _L,), jnp.int32))
    return pl.kernel(
        body,
        out_type=jax.ShapeDtypeStruct((_NW, nbins), jnp.int32),
        mesh=mesh,
        scratch_types=scratch,
        compiler_params=pltpu.CompilerParams(needs_layout_passes=False),
    )


@functools.lru_cache(maxsize=None)
def _sc_kernels():
    p1 = _mk_sc_hist(
        4096,
        lambda ku, idx: lax.shift_right_logical(ku, 20),
        None, 0)
    p2 = _mk_sc_hist(
        4096,
        lambda ku, idx: lax.shift_right_logical(ku, 8) & np.int32(0xFFF),
        lambda ku, idx, pv:
            lax.shift_right_logical(ku, 20) == pv[pl.ds(0, 16)],
        1)
    p3 = _mk_sc_hist(
        256,
        lambda ku, idx: ku & np.int32(0xFF),
        lambda ku, idx, pv:
            lax.shift_right_logical(ku, 8) == pv[pl.ds(0, 16)],
        1)
    i1 = _mk_sc_hist(
        2048,
        lambda ku, idx: lax.shift_right_logical(idx, 11),
        lambda ku, idx, pv: ku == pv[pl.ds(0, 16)],
        1)
    i2 = _mk_sc_hist(
        2048,
        lambda ku, idx: idx & np.int32(0x7FF),
        lambda ku, idx, pv: (ku == pv[pl.ds(0, 16)])
        & (lax.shift_right_logical(idx, 11) == pv[pl.ds(16, 16)]),
        2)
    return p1, p2, p3, i1, i2


def _mk_select(nbins, descending):
    rows = nbins // 128

    def body(kr_ref, h_ref, o_ref):
        hm = jnp.sum(h_ref[...], axis=0).reshape(rows, 128)
        cc = lax.broadcasted_iota(jnp.int32, (128, 128), 0)
        cr = lax.broadcasted_iota(jnp.int32, (128, 128), 1)
        upper = (cc <= cr).astype(jnp.float32)
        ri = lax.broadcasted_iota(jnp.int32, (rows, rows), 0)
        rj = lax.broadcasted_iota(jnp.int32, (rows, rows), 1)
        lstrict = (ri > rj).astype(jnp.float32)
        cincl = jnp.zeros((rows, 128), jnp.int32)
        for k in range(3):
            part = ((hm >> (8 * k)) & np.int32(0xFF)).astype(jnp.float32)
            winp = jnp.dot(part, upper, preferred_element_type=jnp.float32)
            rowtot = winp[:, 127:128].astype(jnp.int32)
            rt_lo = (rowtot & np.int32(0xFF)).astype(jnp.float32)
            rt_hi = (rowtot >> 8).astype(jnp.float32)
            excl = (jnp.dot(lstrict, rt_lo,
                            preferred_element_type=jnp.float32)
                    .astype(jnp.int32)
                    + (jnp.dot(lstrict, rt_hi,
                               preferred_element_type=jnp.float32)
                       .astype(jnp.int32) << 8))
            cincl = cincl + ((winp.astype(jnp.int32) + excl) << (8 * k))
        total = cincl[rows - 1, 127]
        if descending:
            before = total - cincl
        else:
            before = cincl - hm
        kr = kr_ref[0]
        pred = (before < kr) & (before + hm >= kr)
        binidx = (lax.broadcasted_iota(jnp.int32, (rows, 128), 0) * 128
                  + lax.broadcasted_iota(jnp.int32, (rows, 128), 1))
        zero = jnp.zeros((rows, 128), jnp.int32)
        bstar = jnp.sum(jnp.where(pred, binidx, zero))
        bcount = jnp.sum(jnp.where(pred, before, zero))
        bsize = jnp.sum(jnp.where(pred, hm, zero))
        rowsel = lax.broadcasted_iota(jnp.int32, (8, 128), 0)
        out = jnp.where(rowsel == 0, bstar,
                        jnp.where(rowsel == 1, bcount, bsize))
        o_ref[...] = out

    return pl.pallas_call(
        body,
        in_specs=[
            pl.BlockSpec(memory_space=pltpu.SMEM),
            pl.BlockSpec((_NW, nbins), lambda: (0, 0)),
        ],
        out_specs=pl.BlockSpec((8, 128), lambda: (0, 0)),
        out_shape=jax.ShapeDtypeStruct((8, 128), jnp.int32),
    )


_sel_desc_4096 = _mk_select(4096, True)
_sel_desc_256 = _mk_select(256, True)
_sel_asc_2048 = _mk_select(2048, False)


def _mask_body(sel_ref, x_ref, o_ref):
    tt = sel_ref[0]
    q = sel_ref[1]
    ttm = tt ^ _MIN32
    blk = pl.program_id(0)
    x = x_ref[...]
    ku = _sortable(lax.bitcast_convert_type(x, jnp.int32))
    row = lax.broadcasted_iota(jnp.int32, (8, _X), 0) + blk * 8
    col = lax.broadcasted_iota(jnp.int32, (8, _X), 1)
    idx = row * _X + col
    keep = ((ku ^ _MIN32) > ttm) | ((ku == tt) & (idx <= q))
    o_ref[...] = x * keep.astype(jnp.float32)


_mask_call = pl.pallas_call(
    _mask_body,
    grid=(_B // 8,),
    in_specs=[
        pl.BlockSpec(memory_space=pltpu.SMEM),
        pl.BlockSpec((8, _X), lambda i: (i, 0)),
    ],
    out_specs=pl.BlockSpec((8, _X), lambda i: (i, 0)),
    out_shape=jax.ShapeDtypeStruct((_B, _X), jnp.float32),
)


def _splat(v):
    return jnp.full((_L,), v, jnp.int32)


def kernel(input_BX):
    _sc_pass1, _sc_pass2, _sc_pass3, _sc_passI1, _sc_passI2 = _sc_kernels()
    xflat = lax.bitcast_convert_type(input_BX, jnp.int32).reshape(-1)

    h1 = _sc_pass1(xflat)
    s1 = _sel_desc_4096(jnp.array([_TOPK], jnp.int32), h1)
    b1, r1 = s1[0, 0], s1[1, 0]
    kr2 = jnp.int32(_TOPK) - r1

    h2 = _sc_pass2(xflat, _splat(b1))
    s2 = _sel_desc_4096(kr2.reshape(1), h2)
    b2, r2 = s2[0, 0], s2[1, 0]
    kr3 = kr2 - r2
    p24 = (b1 << 12) | b2

    h3 = _sc_pass3(xflat, _splat(p24))
    s3 = _sel_desc_256(kr3.reshape(1), h3)
    b3, r3, m3 = s3[0, 0], s3[1, 0], s3[2, 0]
    need = kr3 - r3
    tt = (p24 << 8) | b3

    def no_split():
        return jnp.int32(_N)

    def split():
        hi1 = _sc_passI1(xflat, _splat(tt))
        a1 = _sel_asc_2048(need.reshape(1), hi1)
        c1, below1 = a1[0, 0], a1[1, 0]
        need2 = need - below1
        hi2 = _sc_passI2(xflat, jnp.concatenate([_splat(tt), _splat(c1)]))
        a2 = _sel_asc_2048(need2.reshape(1), hi2)
        c2 = a2[0, 0]
        return (c1 << 11) | c2

    q = lax.cond(need == m3, no_split, split)
    return _mask_call(jnp.stack([tt, q]), input_BX)
